# baseline (speedup 1.0000x reference)
import functools

import jax
import jax.numpy as jnp
from jax import lax
from jax.experimental import pallas as pl
from jax.experimental.pallas import tpu as pltpu
from jax.experimental.pallas import tpu_sc as plsc

_B = 16384
_V = 100000
_D = 64
_NBLK = 782
_CAPB = 64
_NBUCK = 25
_DUMP = _B


def _embed(c, tableT):
    info = plsc.get_sparse_core_info()
    NC, NS = info.num_cores, info.num_subcores
    mesh = plsc.VectorSubcoreMesh(core_axis_name="c", subcore_axis_name="s")

    @functools.partial(
        pl.kernel,
        mesh=mesh,
        out_type=jax.ShapeDtypeStruct((_B + 16, 128), jnp.float32),
        scratch_types=[
            pltpu.VMEM((_B,), jnp.int32),
            pltpu.VMEM((_NBUCK * 16 * _CAPB,), jnp.int32),
            pltpu.VMEM((_NBUCK * 16,), jnp.int32),
            pltpu.VMEM((_B,), jnp.int32),
            pltpu.VMEM((2, _D, 128), jnp.float32),
            pltpu.VMEM((4, 16, 128), jnp.float32),
            pltpu.SemaphoreType.DMA((2,)),
            pltpu.SemaphoreType.DMA((4,)),
        ],
        compiler_params=pltpu.CompilerParams(needs_layout_passes=False, disable_bounds_checks=True),
    )
    def k(c_hbm, tabT_hbm, out_hbm, c_v, bins, cnts, ovf, bbuf, rbuf, bsem, csem):
        t = lax.axis_index("s") * NC + lax.axis_index("c")
        iota = lax.iota(jnp.int32, 16)
        nb = jnp.where(t <= 13, _NBUCK, _NBUCK - 1)

        def fire_block(j, r):
            vb = t + 32 * j
            pltpu.make_async_copy(
                tabT_hbm.at[:, pl.ds(vb * 128, 128)], bbuf.at[r], bsem.at[r]
            ).start()

        def wait_block(j, r):
            vb = t + 32 * j
            pltpu.make_async_copy(
                tabT_hbm.at[:, pl.ds(vb * 128, 128)], bbuf.at[r], bsem.at[r]
            ).wait()

        pltpu.make_async_copy(c_hbm, c_v, csem.at[0]).start()
        fire_block(0, 0)

        @pl.when(nb > 1)
        def _():
            fire_block(1, 1)

        def zi(g, _):
            cnts[pl.ds(g * 16, 16)] = jnp.zeros((16,), jnp.int32)
            return 0

        lax.fori_loop(0, _NBUCK, zi, 0)
        pltpu.make_async_copy(c_hbm, c_v, csem.at[0]).wait()

        def ga(g, ocnt):
            idx16 = c_v[pl.ds(g * 16, 16)]
            blk = idx16 >> 7
            mine = (blk & 31) == t
            buck = idx16 >> 12
            packed = ((iota + g * 16) << 17) | idx16
            caddr = buck * 16 + iota
            slot = plsc.load_gather(cnts, [caddr])
            over = slot >= _CAPB
            okm = jnp.logical_and(mine, jnp.logical_not(over))
            dest = buck * (16 * _CAPB) + iota * _CAPB + slot
            plsc.store_scatter(bins, [dest], packed, mask=okm)
            plsc.store_scatter(cnts, [caddr], slot + 1, mask=okm)
            ovm = jnp.logical_and(mine, over)

            def slow(oc):
                for l in range(16):
                    m_l = jnp.logical_and(ovm, iota == l)
                    plsc.store_scatter(ovf, [jnp.full((16,), oc, jnp.int32)],
                                       packed, mask=m_l)
                    oc = oc + jnp.sum(m_l.astype(jnp.int32))
                return oc

            return lax.cond(jnp.any(ovm), slow, lambda oc: oc, ocnt)

        ocnt = lax.fori_loop(0, _B // 16, ga, 0)
        novf_g = (ocnt + 15) >> 4

        def extract_chunk(r, active, w16, fired):
            idxv = w16 & 0x1FFFF
            posv = w16 >> 17
            colv = idxv & 127
            q = fired & 3

            @pl.when(fired >= 4)
            def _():
                pltpu.make_async_copy(
                    rbuf.at[q], out_hbm.at[_DUMP + iota], csem.at[q]
                ).wait()

            for d in range(_D):
                dv = jnp.full((16,), d, jnp.int32)
                v = plsc.load_gather(bbuf.at[r], [dv, colv])
                plsc.store_scatter(rbuf.at[q], [iota, dv], v)
            pos_f = jnp.where(active, posv, _DUMP + iota)
            pltpu.make_async_copy(rbuf.at[q], out_hbm.at[pos_f], csem.at[q]).start()
            return fired + 1

        def bk(k2, fired):
            r = k2 & 1
            wait_block(k2, r)
            vb = t + 32 * k2
            cnt16 = plsc.load_gather(cnts, [k2 * 16 + iota])
            maxc = jnp.max(cnt16)

            def sc(s, fired):
                active = cnt16 > s
                w16 = plsc.load_gather(bins, [k2 * (16 * _CAPB) + iota * _CAPB + s])
                return extract_chunk(r, active, w16, fired)

            fired = lax.fori_loop(0, maxc, sc, fired)

            def so(g, fired):
                w16 = plsc.load_gather(ovf, [g * 16 + iota])
                valid = (g * 16 + iota) < ocnt
                active = jnp.logical_and(valid, ((w16 & 0x1FFFF) >> 7) == vb)
                return extract_chunk(r, active, w16, fired)

            fired = lax.fori_loop(0, novf_g, so, fired)

            @pl.when(k2 + 2 < nb)
            def _():
                fire_block(k2 + 2, r)

            return fired

        fired = lax.fori_loop(0, nb, bk, 0)

        for q in range(4):
            @pl.when(fired > q)
            def _():
                pltpu.make_async_copy(
                    rbuf.at[q], out_hbm.at[_DUMP + iota], csem.at[q]
                ).wait()

    return k(c, tableT)


def kernel(c, table):
    out = _embed(c.astype(jnp.int32), table.T)
    return out[:_B, :_D]


# speedup vs baseline: 1.0058x; 1.0058x over previous
import functools

import jax
import jax.numpy as jnp
from jax import lax
from jax.experimental import pallas as pl
from jax.experimental.pallas import tpu as pltpu
from jax.experimental.pallas import tpu_sc as plsc

_B = 16384
_V = 100000
_D = 64
_NBLK = 782
_CAPB = 64
_NBUCK = 25
_DUMP = _B


def _embed(c, tableT):
    info = plsc.get_sparse_core_info()
    NC, NS = info.num_cores, info.num_subcores
    mesh = plsc.VectorSubcoreMesh(core_axis_name="c", subcore_axis_name="s")

    @functools.partial(
        pl.kernel,
        mesh=mesh,
        out_type=jax.ShapeDtypeStruct((_B + 16, 128), jnp.float32),
        scratch_types=[
            pltpu.VMEM((_B,), jnp.int32),
            pltpu.VMEM((_NBUCK * 16 * _CAPB,), jnp.int32),
            pltpu.VMEM((_NBUCK * 16,), jnp.int32),
            pltpu.VMEM((_B,), jnp.int32),
            pltpu.VMEM((2, _D, 128), jnp.float32),
            pltpu.VMEM((4, 16, 128), jnp.float32),
            pltpu.SemaphoreType.DMA((2,)),
            pltpu.SemaphoreType.DMA((4,)),
        ],
        compiler_params=pltpu.CompilerParams(needs_layout_passes=False, disable_bounds_checks=True),
    )
    def k(c_hbm, tabT_hbm, out_hbm, c_v, bins, cnts, ovf, bbuf, rbuf, bsem, csem):
        t = lax.axis_index("s") * NC + lax.axis_index("c")
        iota = lax.iota(jnp.int32, 16)
        nb = jnp.where(t <= 13, _NBUCK, _NBUCK - 1)

        def fire_block(j, r):
            vb = t + 32 * j
            pltpu.make_async_copy(
                tabT_hbm.at[:, pl.ds(vb * 128, 128)], bbuf.at[r], bsem.at[r]
            ).start()

        def wait_block(j, r):
            vb = t + 32 * j
            pltpu.make_async_copy(
                tabT_hbm.at[:, pl.ds(vb * 128, 128)], bbuf.at[r], bsem.at[r]
            ).wait()

        pltpu.make_async_copy(c_hbm, c_v, csem.at[0]).start()
        fire_block(0, 0)

        @pl.when(nb > 1)
        def _():
            fire_block(1, 1)

        def zi(g, _):

            cnts[pl.ds(g * 16, 16)] = jnp.zeros((16,), jnp.int32)
            return 0

        lax.fori_loop(0, _NBUCK, zi, 0)
        pltpu.make_async_copy(c_hbm, c_v, csem.at[0]).wait()

        def ga(g, ocnt):
            idx16 = c_v[pl.ds(g * 16, 16)]
            blk = idx16 >> 7
            mine = (blk & 31) == t
            buck = idx16 >> 12
            packed = ((iota + g * 16) << 17) | idx16
            caddr = buck * 16 + iota
            slot = plsc.load_gather(cnts, [caddr])
            over = slot >= _CAPB
            okm = jnp.logical_and(mine, jnp.logical_not(over))
            dest = buck * (16 * _CAPB) + iota * _CAPB + slot
            plsc.store_scatter(bins, [dest], packed, mask=okm)
            plsc.store_scatter(cnts, [caddr], slot + 1, mask=okm)
            ovm = jnp.logical_and(mine, over)

            def slow(oc):
                for l in range(16):
                    m_l = jnp.logical_and(ovm, iota == l)
                    plsc.store_scatter(ovf, [jnp.full((16,), oc, jnp.int32)],
                                       packed, mask=m_l)
                    oc = oc + jnp.sum(m_l.astype(jnp.int32))
                return oc

            return lax.cond(jnp.any(ovm), slow, lambda oc: oc, ocnt)

        with jax.named_scope("phaseA_scan"):
            ocnt = lax.fori_loop(0, _B // 16, ga, 0)
        novf_g = (ocnt + 15) >> 4

        def extract_chunk(r, active, w16, fired):
            idxv = w16 & 0x1FFFF
            posv = w16 >> 17
            colv = idxv & 127
            q = fired & 3

            @pl.when(fired >= 4)
            def _():
                pltpu.make_async_copy(
                    rbuf.at[q], out_hbm.at[_DUMP + iota], csem.at[q]
                ).wait()

            for d in range(_D):
                dv = jnp.full((16,), d, jnp.int32)
                v = plsc.load_gather(bbuf.at[r], [dv, colv])
                plsc.store_scatter(rbuf.at[q], [iota, dv], v)
            pos_f = jnp.where(active, posv, _DUMP + iota)
            pltpu.make_async_copy(rbuf.at[q], out_hbm.at[pos_f], csem.at[q]).start()
            return fired + 1

        def bk(k2, fired):
            r = k2 & 1
            with jax.named_scope("blockwait"):
                wait_block(k2, r)
            vb = t + 32 * k2
            cnt16 = plsc.load_gather(cnts, [k2 * 16 + iota])
            maxc = jnp.max(cnt16)

            def sc(s, fired):
                active = cnt16 > s
                w16 = plsc.load_gather(bins, [k2 * (16 * _CAPB) + iota * _CAPB + s])
                return extract_chunk(r, active, w16, fired)

            with jax.named_scope("extract"):
                fired = lax.fori_loop(0, maxc, sc, fired)

            def so(g, fired):
                w16 = plsc.load_gather(ovf, [g * 16 + iota])
                valid = (g * 16 + iota) < ocnt
                active = jnp.logical_and(valid, ((w16 & 0x1FFFF) >> 7) == vb)
                return extract_chunk(r, active, w16, fired)

            fired = lax.fori_loop(0, novf_g, so, fired)

            @pl.when(k2 + 2 < nb)
            def _():
                fire_block(k2 + 2, r)

            return fired

        with jax.named_scope("phaseB_blocks"):
            fired = lax.fori_loop(0, nb, bk, 0)

        for q in range(4):
            @pl.when(fired > q)
            def _():
                pltpu.make_async_copy(
                    rbuf.at[q], out_hbm.at[_DUMP + iota], csem.at[q]
                ).wait()

    return k(c, tableT)


def kernel(c, table):
    out = _embed(c.astype(jnp.int32), table.T)
    return out[:_B, :_D]


# speedup vs baseline: 1.6470x; 1.6375x over previous
import functools

import jax
import jax.numpy as jnp
from jax import lax
from jax.experimental import pallas as pl
from jax.experimental.pallas import tpu as pltpu
from jax.experimental.pallas import tpu_sc as plsc


def _gather_pairs(c, t2):
    B, = c.shape
    R, W = t2.shape
    info = plsc.get_sparse_core_info()
    NC, NS = info.num_cores, info.num_subcores
    NW = NC * NS
    b_per_w = B // NW
    mesh = plsc.VectorSubcoreMesh(core_axis_name="c", subcore_axis_name="s")

    @functools.partial(
        pl.kernel,
        mesh=mesh,
        out_type=jax.ShapeDtypeStruct((B, W), jnp.float32),
        scratch_types=[
            pltpu.VMEM((b_per_w,), jnp.int32),
            pltpu.VMEM((b_per_w, W), jnp.float32),
            pltpu.SemaphoreType.DMA,
        ],
    )
    def k(c_hbm, t2_hbm, out_hbm, idx_v, rows_v, sem):
        wid = lax.axis_index("s") * NC + lax.axis_index("c")
        base = wid * b_per_w
        pltpu.sync_copy(c_hbm.at[pl.ds(base, b_per_w)], idx_v)

        def half(g, _):
            v = idx_v[pl.ds(g * 16, 16)]
            idx_v[pl.ds(g * 16, 16)] = v >> 1
            return 0

        lax.fori_loop(0, b_per_w // 16, half, 0)
        pltpu.async_copy(t2_hbm.at[idx_v], rows_v, sem).wait()
        pltpu.sync_copy(rows_v, out_hbm.at[pl.ds(base, b_per_w)])

    return k(c, t2)


def kernel(c, table):
    V, D = table.shape
    ci = c.astype(jnp.int32)
    t2 = table.reshape(V // 2, 2 * D)
    g = _gather_pairs(ci, t2)
    odd = (ci & 1)[:, None] == 1
    return jnp.where(odd, g[:, D:], g[:, :D])


# speedup vs baseline: 2.0587x; 1.2500x over previous
import functools

import jax
import jax.numpy as jnp
from jax import lax
from jax.experimental import pallas as pl
from jax.experimental.pallas import tpu as pltpu
from jax.experimental.pallas import tpu_sc as plsc


def _gather_rows(c, table):
    B, = c.shape
    V, D = table.shape
    info = plsc.get_sparse_core_info()
    NC, NS = info.num_cores, info.num_subcores
    NW = NC * NS
    b_per_w = B // NW
    mesh = plsc.VectorSubcoreMesh(core_axis_name="c", subcore_axis_name="s")

    @functools.partial(
        pl.kernel,
        mesh=mesh,
        out_type=jax.ShapeDtypeStruct((B, D), jnp.float32),
        scratch_types=[
            pltpu.VMEM((b_per_w,), jnp.int32),
            pltpu.VMEM((b_per_w, D), jnp.float32),
            pltpu.SemaphoreType.DMA,
        ],
    )
    def k(c_hbm, table_hbm, out_hbm, idx_v, rows_v, sem):
        wid = lax.axis_index("s") * NC + lax.axis_index("c")
        base = wid * b_per_w
        pltpu.sync_copy(c_hbm.at[pl.ds(base, b_per_w)], idx_v)
        pltpu.async_copy(table_hbm.at[idx_v], rows_v, sem).wait()
        pltpu.sync_copy(rows_v, out_hbm.at[pl.ds(base, b_per_w)])

    return k(c, table)


def kernel(c, table):
    D = table.shape[1]
    tpad = jnp.pad(table, ((0, 0), (0, 128 - D)))
    out = _gather_rows(c.astype(jnp.int32), tpad)
    return out[:, :D]
